# baseline (speedup 1.0000x reference)
import functools

import jax
import jax.numpy as jnp
from jax import lax
from jax.experimental import pallas as pl
from jax.experimental.pallas import tpu as pltpu
from jax.experimental.pallas import tpu_sc as plsc

_E = 64
_TILE = 32
_NT = 96
_NW = 32
_RING = 2


def _route_body(x_ref, wr_ref, pos1_ref, pos2_ref, tab_ref, q_ref):
    T = x_ref.shape[0]
    PAD = q_ref.shape[1]
    x = x_ref[...]
    logits = lax.dot_general(x, wr_ref[...], (((1,), (1,)), ((), ())),
                             preferred_element_type=jnp.float32)
    m = jnp.max(logits, axis=1, keepdims=True)
    p = jnp.exp(logits - m)
    probs = p / jnp.sum(p, axis=1, keepdims=True)
    lane = lax.broadcasted_iota(jnp.int32, (T, _E), 1)
    m1 = jnp.max(probs, axis=1, keepdims=True)
    i1 = jnp.min(jnp.where(probs == m1, lane, _E), axis=1, keepdims=True)
    probs2 = jnp.where(lane == i1, -jnp.inf, probs)
    m2 = jnp.max(probs2, axis=1, keepdims=True)
    i2 = jnp.min(jnp.where(probs2 == m2, lane, _E), axis=1, keepdims=True)
    s = m1 + m2
    w1 = m1 / s
    w2 = m2 / s

    M1 = (lane == i1).astype(jnp.float32)
    M2 = (lane == i2).astype(jnp.float32)
    cnt1 = jnp.sum(M1, axis=0, keepdims=True)
    cnt2 = jnp.sum(M2, axis=0, keepdims=True)
    cnt = cnt1 + cnt2
    pcnt = jnp.ceil(cnt / _TILE) * _TILE
    ea = lax.broadcasted_iota(jnp.int32, (_E, _E), 0)
    eb = lax.broadcasted_iota(jnp.int32, (_E, _E), 1)
    U = (ea <= eb).astype(jnp.float32)
    pcum = lax.dot_general(pcnt, U, (((1,), (0,)), ((), ())),
                           preferred_element_type=jnp.float32)
    pex = pcum - pcnt
    ta = lax.broadcasted_iota(jnp.int32, (T, T), 0)
    tb = lax.broadcasted_iota(jnp.int32, (T, T), 1)
    S = (tb < ta).astype(jnp.float32)
    C1 = lax.dot_general(S, M1, (((1,), (0,)), ((), ())),
                         preferred_element_type=jnp.float32)
    C2 = lax.dot_general(S, M2, (((1,), (0,)), ((), ())),
                         preferred_element_type=jnp.float32)
    r1 = jnp.sum(C1 * M1, axis=1, keepdims=True)
    r2 = jnp.sum(C2 * M2, axis=1, keepdims=True)
    pos1 = jnp.sum(pex * M1, axis=1, keepdims=True) + r1
    pos2 = jnp.sum((pex + cnt1) * M2, axis=1, keepdims=True) + r2
    pos1i = jnp.round(pos1).astype(jnp.int32)
    pos2i = jnp.round(pos2).astype(jnp.int32)
    pos1_ref[...] = pos1i
    pos2_ref[...] = pos2i

    ts = jnp.round(pex / _TILE).astype(jnp.int32)
    nt = jnp.round(pcnt / _TILE).astype(jnp.int32)
    cs = jnp.round(cnt).astype(jnp.int32)
    tab_ref[0:8, :] = jnp.broadcast_to(ts, (8, _E))
    tab_ref[8:16, :] = jnp.broadcast_to(nt, (8, _E))
    tab_ref[16:24, :] = jnp.broadcast_to(cs, (8, _E))

    slot = lax.broadcasted_iota(jnp.int32, (T, PAD), 1)
    occ1 = (slot == pos1i).astype(jnp.float32)
    occ2 = (slot == pos2i).astype(jnp.float32)
    q_ref[...] = occ1 * w1 + occ2 * w2


def _expert_body(tab_ref, q_ref, gx_ref, wg_hbm, wu_hbm, wd_hbm, out_ref,
                 wgb, wub, wdb, yv, sems):
    K = _RING
    PAD = yv.shape[0]

    def issue(eidx, b):
        pltpu.make_async_copy(wg_hbm.at[eidx], wgb.at[b], sems.at[b, 0]).start()
        pltpu.make_async_copy(wu_hbm.at[eidx], wub.at[b], sems.at[b, 1]).start()
        pltpu.make_async_copy(wd_hbm.at[eidx], wdb.at[b], sems.at[b, 2]).start()

    for b in range(K):
        issue(b, b)

    def zero_body(i, _):
        yv[pl.ds(i * _TILE, _TILE), :] = jnp.zeros((_TILE, yv.shape[1]),
                                                   jnp.float32)
        return 0

    lax.fori_loop(0, PAD // _TILE, zero_body, 0)

    def step(eo, _):
        for b in range(K):
            e = eo * K + b
            pltpu.make_async_copy(wg_hbm.at[e], wgb.at[b], sems.at[b, 0]).wait()
            pltpu.make_async_copy(wu_hbm.at[e], wub.at[b], sems.at[b, 1]).wait()
            pltpu.make_async_copy(wd_hbm.at[e], wdb.at[b], sems.at[b, 2]).wait()
            wg = wgb[b]
            wu = wub[b]
            wd = wdb[b]
            start = tab_ref[0, e]
            cnt = tab_ref[16, e]
            riota = lax.broadcasted_iota(jnp.int32, (_TILE, 1), 0)

            def tile_body(k, _, wg=wg, wu=wu, wd=wd, start=start, cnt=cnt,
                          riota=riota):
                row0 = (start + k) * _TILE
                gx = gx_ref[pl.ds(row0, _TILE), :]
                g = lax.dot_general(gx, wg, (((1,), (1,)), ((), ())),
                                    preferred_element_type=jnp.float32)
                u = lax.dot_general(gx, wu, (((1,), (1,)), ((), ())),
                                    preferred_element_type=jnp.float32)
                h = (g * jax.nn.sigmoid(g)) * u
                o = lax.dot_general(h, wd, (((1,), (1,)), ((), ())),
                                    preferred_element_type=jnp.float32)
                o = jnp.where(riota < (cnt - k * _TILE), o, 0.0)
                yv[pl.ds(row0, _TILE), :] = o
                return 0

            lax.fori_loop(0, tab_ref[8, e], tile_body, 0)

            @pl.when(e + K < _E)
            def _():
                issue(e + K, b)

        return 0

    lax.fori_loop(0, _E // K, step, 0)

    out_ref[...] = lax.dot_general(q_ref[...], yv[...],
                                   (((1,), (0,)), ((), ())),
                                   preferred_element_type=jnp.float32)


def _make_scatter(T, D, PAD):
    ntok = T // _NW
    mesh = plsc.VectorSubcoreMesh(core_axis_name="c", subcore_axis_name="s")

    @functools.partial(
        pl.kernel, mesh=mesh,
        out_type=jax.ShapeDtypeStruct((PAD, D), jnp.float32),
        scratch_types=[
            pltpu.VMEM((ntok,), jnp.int32),
            pltpu.VMEM((ntok,), jnp.int32),
            pltpu.VMEM((ntok, D), jnp.float32),
            pltpu.SemaphoreType.DMA,
            pltpu.SemaphoreType.DMA,
        ],
    )
    def scatter_k(x_hbm, p1_hbm, p2_hbm, gx_hbm, i1v, i2v, rows, sem1, sem2):
        wid = lax.axis_index("s") * 2 + lax.axis_index("c")
        base = wid * ntok
        pltpu.sync_copy(p1_hbm.at[pl.ds(base, ntok)], i1v)
        pltpu.sync_copy(p2_hbm.at[pl.ds(base, ntok)], i2v)
        pltpu.sync_copy(x_hbm.at[pl.ds(base, ntok)], rows)
        c1 = pltpu.async_copy(rows, gx_hbm.at[i1v], sem1)
        c2 = pltpu.async_copy(rows, gx_hbm.at[i2v], sem2)
        c1.wait()
        c2.wait()

    return scatter_k


def kernel(hidden_states, W_router, W_gate, W_up, W_down):
    b, s, d = hidden_states.shape
    x = hidden_states.reshape(-1, d)
    T = x.shape[0]
    E, FF = W_gate.shape[0], W_gate.shape[1]
    PAD = _NT * _TILE

    pos1, pos2, tab, q = pl.pallas_call(
        _route_body,
        grid=(1,),
        in_specs=[
            pl.BlockSpec((T, d), lambda i: (0, 0)),
            pl.BlockSpec((E, d), lambda i: (0, 0)),
        ],
        out_specs=[
            pl.BlockSpec((T, 1), lambda i: (0, 0)),
            pl.BlockSpec((T, 1), lambda i: (0, 0)),
            pl.BlockSpec((24, E), lambda i: (0, 0)),
            pl.BlockSpec((T, PAD), lambda i: (0, 0)),
        ],
        out_shape=[
            jax.ShapeDtypeStruct((T, 1), jnp.int32),
            jax.ShapeDtypeStruct((T, 1), jnp.int32),
            jax.ShapeDtypeStruct((24, E), jnp.int32),
            jax.ShapeDtypeStruct((T, PAD), jnp.float32),
        ],
    )(x, W_router)
    p1 = pos1.reshape(T)
    p2 = pos2.reshape(T)

    gx = _make_scatter(T, d, PAD)(x, p1, p2)

    out = pl.pallas_call(
        _expert_body,
        grid=(1,),
        in_specs=[
            pl.BlockSpec(memory_space=pltpu.SMEM),
            pl.BlockSpec((T, PAD), lambda i: (0, 0)),
            pl.BlockSpec((PAD, d), lambda i: (0, 0)),
            pl.BlockSpec(memory_space=pl.ANY),
            pl.BlockSpec(memory_space=pl.ANY),
            pl.BlockSpec(memory_space=pl.ANY),
        ],
        out_specs=pl.BlockSpec((T, d), lambda i: (0, 0)),
        out_shape=jax.ShapeDtypeStruct((T, d), jnp.float32),
        scratch_shapes=[
            pltpu.VMEM((_RING, FF, d), jnp.float32),
            pltpu.VMEM((_RING, FF, d), jnp.float32),
            pltpu.VMEM((_RING, d, FF), jnp.float32),
            pltpu.VMEM((PAD, d), jnp.float32),
            pltpu.SemaphoreType.DMA((_RING, 3)),
        ],
    )(tab, q, gx, W_gate, W_up, W_down)
    return out.reshape(b, s, d)


# speedup vs baseline: 1.1531x; 1.1531x over previous
import functools

import jax
import jax.numpy as jnp
from jax import lax
from jax.experimental import pallas as pl
from jax.experimental.pallas import tpu as pltpu
from jax.experimental.pallas import tpu_sc as plsc

_E = 64
_TILE = 32
_NT = 96
_NW = 32
_RING = 3


def _route_body(x_ref, wr_ref, pos1_ref, pos2_ref, tab_ref, q_ref):
    T = x_ref.shape[0]
    PAD = q_ref.shape[1]
    x = x_ref[...]
    logits = lax.dot_general(x, wr_ref[...], (((1,), (1,)), ((), ())),
                             preferred_element_type=jnp.float32)
    m = jnp.max(logits, axis=1, keepdims=True)
    p = jnp.exp(logits - m)
    probs = p / jnp.sum(p, axis=1, keepdims=True)
    lane = lax.broadcasted_iota(jnp.int32, (T, _E), 1)
    m1 = jnp.max(probs, axis=1, keepdims=True)
    i1 = jnp.min(jnp.where(probs == m1, lane, _E), axis=1, keepdims=True)
    probs2 = jnp.where(lane == i1, -jnp.inf, probs)
    m2 = jnp.max(probs2, axis=1, keepdims=True)
    i2 = jnp.min(jnp.where(probs2 == m2, lane, _E), axis=1, keepdims=True)
    s = m1 + m2
    w1 = m1 / s
    w2 = m2 / s

    M1 = (lane == i1).astype(jnp.float32)
    M2 = (lane == i2).astype(jnp.float32)
    cnt1 = jnp.sum(M1, axis=0, keepdims=True)
    cnt2 = jnp.sum(M2, axis=0, keepdims=True)
    cnt = cnt1 + cnt2
    pcnt = jnp.ceil(cnt / _TILE) * _TILE
    ea = lax.broadcasted_iota(jnp.int32, (_E, _E), 0)
    eb = lax.broadcasted_iota(jnp.int32, (_E, _E), 1)
    U = (ea <= eb).astype(jnp.float32)
    pcum = lax.dot_general(pcnt, U, (((1,), (0,)), ((), ())),
                           preferred_element_type=jnp.float32)
    pex = pcum - pcnt
    ta = lax.broadcasted_iota(jnp.int32, (T, T), 0)
    tb = lax.broadcasted_iota(jnp.int32, (T, T), 1)
    S = (tb < ta).astype(jnp.float32)
    C1 = lax.dot_general(S, M1, (((1,), (0,)), ((), ())),
                         preferred_element_type=jnp.float32)
    C2 = lax.dot_general(S, M2, (((1,), (0,)), ((), ())),
                         preferred_element_type=jnp.float32)
    r1 = jnp.sum(C1 * M1, axis=1, keepdims=True)
    r2 = jnp.sum(C2 * M2, axis=1, keepdims=True)
    pos1 = jnp.sum(pex * M1, axis=1, keepdims=True) + r1
    pos2 = jnp.sum((pex + cnt1) * M2, axis=1, keepdims=True) + r2
    pos1i = jnp.round(pos1).astype(jnp.int32)
    pos2i = jnp.round(pos2).astype(jnp.int32)
    pos1_ref[...] = pos1i
    pos2_ref[...] = pos2i

    ts = jnp.round(pex / _TILE).astype(jnp.int32)
    nt = jnp.round(pcnt / _TILE).astype(jnp.int32)
    cs = jnp.round(cnt).astype(jnp.int32)
    tab_ref[0:8, :] = jnp.broadcast_to(ts, (8, _E))
    tab_ref[8:16, :] = jnp.broadcast_to(nt, (8, _E))
    tab_ref[16:24, :] = jnp.broadcast_to(cs, (8, _E))

    slot = lax.broadcasted_iota(jnp.int32, (T, PAD), 1)
    occ1 = (slot == pos1i).astype(jnp.float32)
    occ2 = (slot == pos2i).astype(jnp.float32)
    q_ref[...] = occ1 * w1 + occ2 * w2


def _expert_body(tab_ref, q_ref, gx_ref, wg_hbm, wu_hbm, wd_hbm, out_ref,
                 wgb, wub, wdb, yv, sems):
    K = _RING
    PAD = yv.shape[0]

    def issue(eidx, b):
        pltpu.make_async_copy(wg_hbm.at[eidx], wgb.at[b], sems.at[b, 0]).start()
        pltpu.make_async_copy(wu_hbm.at[eidx], wub.at[b], sems.at[b, 1]).start()
        pltpu.make_async_copy(wd_hbm.at[eidx], wdb.at[b], sems.at[b, 2]).start()

    for b in range(K):
        issue(b, b)

    def zero_body(i, _):
        yv[pl.ds(i * _TILE, _TILE), :] = jnp.zeros((_TILE, yv.shape[1]),
                                                   jnp.float32)
        return 0

    lax.fori_loop(0, PAD // _TILE, zero_body, 0)

    def step(eo, _):
        for b in range(K):
            e = eo * K + b

            @pl.when(e < _E)
            def _(e=e, b=b):
                pltpu.make_async_copy(wg_hbm.at[e], wgb.at[b],
                                      sems.at[b, 0]).wait()
                pltpu.make_async_copy(wu_hbm.at[e], wub.at[b],
                                      sems.at[b, 1]).wait()
                pltpu.make_async_copy(wd_hbm.at[e], wdb.at[b],
                                      sems.at[b, 2]).wait()
                wg = wgb[b]
                wu = wub[b]
                wd = wdb[b]
                start = tab_ref[0, e]
                cnt = tab_ref[16, e]
                riota = lax.broadcasted_iota(jnp.int32, (_TILE, 1), 0)

                def tile_body(k, _, wg=wg, wu=wu, wd=wd, start=start, cnt=cnt,
                              riota=riota):
                    row0 = (start + k) * _TILE
                    gx = gx_ref[pl.ds(row0, _TILE), :]
                    g = lax.dot_general(gx, wg, (((1,), (1,)), ((), ())),
                                        preferred_element_type=jnp.float32)
                    u = lax.dot_general(gx, wu, (((1,), (1,)), ((), ())),
                                        preferred_element_type=jnp.float32)
                    h = (g * jax.nn.sigmoid(g)) * u
                    o = lax.dot_general(h, wd, (((1,), (1,)), ((), ())),
                                        preferred_element_type=jnp.float32)
                    o = jnp.where(riota < (cnt - k * _TILE), o, 0.0)
                    yv[pl.ds(row0, _TILE), :] = o
                    return 0

                lax.fori_loop(0, tab_ref[8, e], tile_body, 0)

                @pl.when(e + K < _E)
                def _():
                    issue(e + K, b)

        return 0

    lax.fori_loop(0, (_E + K - 1) // K, step, 0)

    out_ref[...] = lax.dot_general(q_ref[...], yv[...],
                                   (((1,), (0,)), ((), ())),
                                   preferred_element_type=jnp.float32)


def _make_scatter(T, D, PAD):
    ntok = T // _NW
    mesh = plsc.VectorSubcoreMesh(core_axis_name="c", subcore_axis_name="s")

    @functools.partial(
        pl.kernel, mesh=mesh,
        out_type=jax.ShapeDtypeStruct((PAD, D), jnp.float32),
        scratch_types=[
            pltpu.VMEM((ntok,), jnp.int32),
            pltpu.VMEM((ntok,), jnp.int32),
            pltpu.VMEM((ntok, D), jnp.float32),
            pltpu.SemaphoreType.DMA,
            pltpu.SemaphoreType.DMA,
        ],
    )
    def scatter_k(x_hbm, p1_hbm, p2_hbm, gx_hbm, i1v, i2v, rows, sem1, sem2):
        wid = lax.axis_index("s") * 2 + lax.axis_index("c")
        base = wid * ntok
        pltpu.sync_copy(p1_hbm.at[pl.ds(base, ntok)], i1v)
        pltpu.sync_copy(p2_hbm.at[pl.ds(base, ntok)], i2v)
        pltpu.sync_copy(x_hbm.at[pl.ds(base, ntok)], rows)
        c1 = pltpu.async_copy(rows, gx_hbm.at[i1v], sem1)
        c2 = pltpu.async_copy(rows, gx_hbm.at[i2v], sem2)
        c1.wait()
        c2.wait()

    return scatter_k


def kernel(hidden_states, W_router, W_gate, W_up, W_down):
    b, s, d = hidden_states.shape
    x = hidden_states.reshape(-1, d)
    T = x.shape[0]
    E, FF = W_gate.shape[0], W_gate.shape[1]
    PAD = _NT * _TILE

    pos1, pos2, tab, q = pl.pallas_call(
        _route_body,
        grid=(1,),
        in_specs=[
            pl.BlockSpec((T, d), lambda i: (0, 0)),
            pl.BlockSpec((E, d), lambda i: (0, 0)),
        ],
        out_specs=[
            pl.BlockSpec((T, 1), lambda i: (0, 0)),
            pl.BlockSpec((T, 1), lambda i: (0, 0)),
            pl.BlockSpec((24, E), lambda i: (0, 0)),
            pl.BlockSpec((T, PAD), lambda i: (0, 0)),
        ],
        out_shape=[
            jax.ShapeDtypeStruct((T, 1), jnp.int32),
            jax.ShapeDtypeStruct((T, 1), jnp.int32),
            jax.ShapeDtypeStruct((24, E), jnp.int32),
            jax.ShapeDtypeStruct((T, PAD), jnp.float32),
        ],
    )(x, W_router)
    p1 = pos1.reshape(T)
    p2 = pos2.reshape(T)

    gx = _make_scatter(T, d, PAD)(x, p1, p2)

    out = pl.pallas_call(
        _expert_body,
        grid=(1,),
        in_specs=[
            pl.BlockSpec(memory_space=pltpu.SMEM),
            pl.BlockSpec((T, PAD), lambda i: (0, 0)),
            pl.BlockSpec((PAD, d), lambda i: (0, 0)),
            pl.BlockSpec(memory_space=pl.ANY),
            pl.BlockSpec(memory_space=pl.ANY),
            pl.BlockSpec(memory_space=pl.ANY),
        ],
        out_specs=pl.BlockSpec((T, d), lambda i: (0, 0)),
        out_shape=jax.ShapeDtypeStruct((T, d), jnp.float32),
        scratch_shapes=[
            pltpu.VMEM((_RING, FF, d), jnp.float32),
            pltpu.VMEM((_RING, FF, d), jnp.float32),
            pltpu.VMEM((_RING, d, FF), jnp.float32),
            pltpu.VMEM((PAD, d), jnp.float32),
            pltpu.SemaphoreType.DMA((_RING, 3)),
        ],
    )(tab, q, gx, W_gate, W_up, W_down)
    return out.reshape(b, s, d)
